# Initial kernel scaffold; baseline (speedup 1.0000x reference)
import jax
import jax.numpy as jnp
from jax.experimental import pallas as pl


def kernel(context, y, W_month, W_day, W_hour, W_dow, W_pos):
    raise NotImplementedError("write your pallas kernel here")


# speedup vs baseline: 2.2503x; 2.2503x over previous
import functools

import jax
import jax.numpy as jnp
from jax import lax
from jax.experimental import pallas as pl
from jax.experimental.pallas import tpu as pltpu
from jax.experimental.pallas import tpu_sc as plsc

EMBED = 64
NC = 2
NS = 16
NW = NC * NS
CHUNK = 128
LANES = 16


def _sc_body(per_w, n_chunks, ctx_hbm, wm, wd, wh, ww, wpos, out_hbm,
             ctx_v, idx0, idx1, idx2, idx3, rows0, rows1, rows2, rows3,
             const_v, sem):
    wid = lax.axis_index("s") * NC + lax.axis_index("c")
    lanes = lax.iota(jnp.int32, 16)

    pltpu.sync_copy(wpos.at[pl.ds(0, 1)], const_v.at[pl.ds(0, 1)])
    crow = [const_v[0, pl.ds(l * LANES, LANES)] for l in range(EMBED // LANES)]

    def fill_body(r, carry):
        for l in range(EMBED // LANES):
            const_v[r, pl.ds(l * LANES, LANES)] = crow[l]
        return carry

    lax.fori_loop(1, CHUNK, fill_body, 0)

    idx_refs = [idx0, idx1, idx2, idx3]
    row_refs = [rows0, rows1, rows2, rows3]
    tables = [wm, wd, wh, ww]

    def chunk_body(c, carry):
        base = wid * per_w + c * CHUNK
        pltpu.sync_copy(ctx_hbm.at[pl.ds(base, CHUNK)], ctx_v)
        for t in range(4):
            col = jnp.full((16,), t + 1, jnp.int32)
            for v in range(CHUNK // LANES):
                vals = plsc.load_gather(ctx_v, [lanes + (v * LANES), col])
                idx_refs[t][pl.ds(v * LANES, LANES)] = vals
        copies = [
            pltpu.async_copy(tables[t].at[idx_refs[t]], row_refs[t], sem)
            for t in range(4)
        ]
        for cp in copies:
            cp.wait()
        for t in range(4):
            pltpu.sync_copy(row_refs[t],
                            out_hbm.at[pl.ds(base, CHUNK),
                                       pl.ds(t * EMBED, EMBED)])
        pltpu.sync_copy(const_v,
                        out_hbm.at[pl.ds(base, CHUNK), pl.ds(4 * EMBED, EMBED)])
        return carry

    lax.fori_loop(0, n_chunks, chunk_body, 0)


def kernel(context, y, W_month, W_day, W_hour, W_dow, W_pos):
    del y
    Bc, Nc, _ = context.shape
    M = Bc * Nc
    assert M % (NW * CHUNK) == 0
    per_w = M // NW
    n_chunks = per_w // CHUNK
    ctx = context.reshape(M, 5).astype(jnp.int32)

    mesh = plsc.VectorSubcoreMesh(core_axis_name="c", subcore_axis_name="s")
    f32 = jnp.float32
    run = pl.kernel(
        functools.partial(_sc_body, per_w, n_chunks),
        out_type=jax.ShapeDtypeStruct((M, 5 * EMBED), f32),
        mesh=mesh,
        scratch_types=[
            pltpu.VMEM((CHUNK, 5), jnp.int32),
            pltpu.VMEM((CHUNK,), jnp.int32),
            pltpu.VMEM((CHUNK,), jnp.int32),
            pltpu.VMEM((CHUNK,), jnp.int32),
            pltpu.VMEM((CHUNK,), jnp.int32),
            pltpu.VMEM((CHUNK, EMBED), f32),
            pltpu.VMEM((CHUNK, EMBED), f32),
            pltpu.VMEM((CHUNK, EMBED), f32),
            pltpu.VMEM((CHUNK, EMBED), f32),
            pltpu.VMEM((CHUNK, EMBED), f32),
            pltpu.SemaphoreType.DMA,
        ],
        compiler_params=pltpu.CompilerParams(use_tc_tiling_on_sc=False,
                                             needs_layout_passes=False),
    )
    out = run(ctx, W_month.astype(f32), W_day.astype(f32),
              W_hour.astype(f32), W_dow.astype(f32), W_pos.astype(f32))
    return out.reshape(Bc, Nc, 5 * EMBED)


# speedup vs baseline: 2.3338x; 1.0371x over previous
import functools

import jax
import jax.numpy as jnp
from jax import lax
from jax.experimental import pallas as pl
from jax.experimental.pallas import tpu as pltpu
from jax.experimental.pallas import tpu_sc as plsc

EMBED = 64
NC = 2
NS = 16
NW = NC * NS
CHUNK = 128
LANES = 16
NSETS = 3
CONST_ROWS = 64


def _sc_body(per_w, n_chunks, idx_hbm, wm, wd, wh, ww, wpos, out_hbm,
             idx_v, rows_v, const_v, semg0, semg1, semg2,
             sems0, sems1, sems2):
    wid = lax.axis_index("s") * NC + lax.axis_index("c")
    base_w = wid * per_w
    tables = [wm, wd, wh, ww]
    semg = [semg0, semg1, semg2]
    sems = [sems0, sems1, sems2]

    pltpu.sync_copy(idx_hbm.at[:, pl.ds(base_w, per_w)], idx_v)

    pltpu.sync_copy(wpos.at[pl.ds(0, 1)], const_v.at[pl.ds(0, 1)])
    crow = [const_v[0, pl.ds(l * LANES, LANES)] for l in range(EMBED // LANES)]

    def fill_body(r, carry):
        for l in range(EMBED // LANES):
            const_v[r, pl.ds(l * LANES, LANES)] = crow[l]
        return carry

    lax.fori_loop(1, CONST_ROWS, fill_body, 0)

    g_descs = {}
    s_descs = {}
    for c in range(n_chunks + 2):
        if c < n_chunks:
            s = c % NSETS
            if c - NSETS >= 0:
                for d in s_descs.pop(c - NSETS):
                    d.wait()
            g_descs[c] = [
                pltpu.async_copy(
                    tables[t].at[idx_v.at[t, pl.ds(c * CHUNK, CHUNK)]],
                    rows_v.at[s, t], semg[s])
                for t in range(4)
            ]
        j = c - 2
        if 0 <= j < n_chunks:
            s = j % NSETS
            for d in g_descs.pop(j):
                d.wait()
            base = base_w + j * CHUNK
            descs = [
                pltpu.async_copy(rows_v.at[s, t],
                                 out_hbm.at[pl.ds(base, CHUNK),
                                            pl.ds(t * EMBED, EMBED)],
                                 sems[s])
                for t in range(4)
            ]
            for h in range(CHUNK // CONST_ROWS):
                descs.append(
                    pltpu.async_copy(
                        const_v,
                        out_hbm.at[pl.ds(base + h * CONST_ROWS, CONST_ROWS),
                                   pl.ds(4 * EMBED, EMBED)],
                        sems[s]))
            s_descs[j] = descs
    for j in sorted(s_descs):
        for d in s_descs.pop(j):
            d.wait()


def kernel(context, y, W_month, W_day, W_hour, W_dow, W_pos):
    del y
    Bc, Nc, _ = context.shape
    M = Bc * Nc
    assert M % (NW * CHUNK) == 0
    per_w = M // NW
    n_chunks = per_w // CHUNK
    idxs = context.reshape(M, 5).astype(jnp.int32)[:, 1:5].T

    mesh = plsc.VectorSubcoreMesh(core_axis_name="c", subcore_axis_name="s")
    f32 = jnp.float32
    run = pl.kernel(
        functools.partial(_sc_body, per_w, n_chunks),
        out_type=jax.ShapeDtypeStruct((M, 5 * EMBED), f32),
        mesh=mesh,
        scratch_types=[
            pltpu.VMEM((4, per_w), jnp.int32),
            pltpu.VMEM((NSETS, 4, CHUNK, EMBED), f32),
            pltpu.VMEM((CONST_ROWS, EMBED), f32),
            pltpu.SemaphoreType.DMA,
            pltpu.SemaphoreType.DMA,
            pltpu.SemaphoreType.DMA,
            pltpu.SemaphoreType.DMA,
            pltpu.SemaphoreType.DMA,
            pltpu.SemaphoreType.DMA,
        ],
        compiler_params=pltpu.CompilerParams(use_tc_tiling_on_sc=False,
                                             needs_layout_passes=False),
    )
    out = run(jnp.asarray(idxs), W_month.astype(f32), W_day.astype(f32),
              W_hour.astype(f32), W_dow.astype(f32), W_pos.astype(f32))
    return out.reshape(Bc, Nc, 5 * EMBED)
